# baseline (speedup 1.0000x reference)
import functools

import jax
import jax.numpy as jnp
from jax import lax
from jax.experimental import pallas as pl
from jax.experimental.pallas import tpu as pltpu
from jax.experimental.pallas import tpu_sc as plsc

N = 10000
E = 320000
D = 128
F = 16

NC = 2
NS = 16
NW = NC * NS

EPT = E // NW
G = 32
NFULL = EPT // G
TAIL = EPT - NFULL * G
NTRIP = NFULL // 3
NPAD = 10240
ROWS_PER_SUB = NPAD // NS
LANES = 16


BX = 16000


def _scale_body(f_ref, w_ref, o_ref):
    i = pl.program_id(0)
    s = jnp.sum(f_ref[...], axis=0) * jnp.sum(w_ref[...], axis=0)
    o_ref[pl.ds(i * BX, BX)] = s


def _compute_scale(edge_filter_t, weight_t):
    return pl.pallas_call(
        _scale_body,
        grid=(E // BX,),
        in_specs=[
            pl.BlockSpec((F, BX), lambda i: (0, i)),
            pl.BlockSpec((F, BX), lambda i: (0, i)),
        ],
        out_specs=pl.BlockSpec((E,), lambda i: (0,)),
        out_shape=jax.ShapeDtypeStruct((E,), jnp.float32),
    )(edge_filter_t, weight_t)


def _add_body(p_ref, o_ref):
    o_ref[...] = p_ref[0] + p_ref[1]


def _combine(partial):
    BR = 2000
    return pl.pallas_call(
        _add_body,
        grid=(N // BR,),
        in_specs=[pl.BlockSpec((NC, BR, D), lambda i: (0, i, 0))],
        out_specs=pl.BlockSpec((BR, D), lambda i: (i, 0)),
        out_shape=jax.ShapeDtypeStruct((N, D), jnp.float32),
    )(partial)


def _sc_main(x, scale, src, dst):
    mesh = plsc.VectorSubcoreMesh(core_axis_name="c", subcore_axis_name="s")

    @functools.partial(
        pl.kernel,
        out_type=jax.ShapeDtypeStruct((NC, NPAD, D), jnp.float32),
        mesh=mesh,
        scratch_types=[
            pltpu.VMEM((EPT,), jnp.int32),
            pltpu.VMEM((EPT,), jnp.int32),
            pltpu.VMEM((EPT,), jnp.float32),
            pltpu.VMEM((3, G, D), jnp.float32),
            pltpu.VMEM((3, G), jnp.int32),
            pltpu.VMEM((TAIL,), jnp.int32),
            pltpu.VMEM_SHARED((NPAD, D), jnp.float32),
            pltpu.SemaphoreType.DMA,
            pltpu.SemaphoreType.DMA,
            pltpu.SemaphoreType.DMA,
            pltpu.SemaphoreType.DMA,
            pltpu.SemaphoreType.DMA,
            pltpu.SemaphoreType.DMA,
        ],
    )
    def body(x_hbm, scale_hbm, src_hbm, dst_hbm, out_hbm,
             src_v, dst_v, scale_v, rows_v, dstg_v, tdst_v, acc,
             gsem0, gsem1, gsem2, ssem0, ssem1, ssem2):
        cid = lax.axis_index("c")
        sid = lax.axis_index("s")
        wid = cid * NS + sid
        base = wid * EPT
        gsems = [gsem0, gsem1, gsem2]
        ssems = [ssem0, ssem1, ssem2]

        pltpu.sync_copy(src_hbm.at[pl.ds(base, EPT)], src_v)
        pltpu.sync_copy(dst_hbm.at[pl.ds(base, EPT)], dst_v)
        pltpu.sync_copy(scale_hbm.at[pl.ds(base, EPT)], scale_v)

        zero = jnp.zeros((LANES,), jnp.float32)

        def zrow(i, carry):
            for q in range(D // LANES):
                rows_v[0, i, pl.ds(q * LANES, LANES)] = zero
            return carry

        lax.fori_loop(0, G, zrow, 0)
        for k in range(ROWS_PER_SUB // G):
            pltpu.sync_copy(
                rows_v.at[0],
                acc.at[pl.ds(sid * ROWS_PER_SUB + k * G, G)])
        plsc.subcore_barrier()

        def issue_gather(j, b):
            return pltpu.async_copy(
                x_hbm.at[src_v.at[pl.ds(j * G, G)]], rows_v.at[b], gsems[b])

        def wait_gather(b):
            pltpu.make_async_copy(
                x_hbm.at[src_v.at[pl.ds(0, G)]], rows_v.at[b],
                gsems[b]).wait()

        def issue_scatter(b):
            return pltpu.async_copy(
                rows_v.at[b], acc.at[dstg_v.at[b]], ssems[b], add=True)

        def wait_scatter(b):
            pltpu.make_async_copy(
                rows_v.at[b], acc.at[dstg_v.at[b]], ssems[b]).wait()

        def stage_dst(j, b):
            for g in range(G // LANES):
                dstg_v[b, pl.ds(g * LANES, LANES)] = (
                    dst_v[pl.ds(j * G + g * LANES, LANES)])

        def compute(j, b):
            for g in range(G // LANES):
                s16 = scale_v[pl.ds(j * G + g * LANES, LANES)]
                for t in range(LANES):
                    e = g * LANES + t
                    s = s16[t]
                    for q in range(D // LANES):
                        sl = pl.ds(q * LANES, LANES)
                        rows_v[b, e, sl] = rows_v[b, e, sl] * s

        for b in range(2):
            stage_dst(b, b)
            issue_gather(b, b)

        def triple_body(p, carry):
            j0 = p * 3
            for b in range(3):
                j = j0 + b
                wait_gather(b)
                compute(j, b)
                issue_scatter(b)

                bn = (b + 2) % 3
                @pl.when(j > 0)
                def _():
                    wait_scatter(bn)

                @pl.when(j + 2 < NFULL)
                def _():
                    stage_dst(j + 2, bn)
                    issue_gather(j + 2, bn)
            return carry

        lax.fori_loop(0, NTRIP, triple_body, 0)

        wait_scatter(2)

        tdst_v[...] = dst_v[pl.ds(NFULL * G, TAIL)]
        pltpu.async_copy(
            x_hbm.at[src_v.at[pl.ds(NFULL * G, TAIL)]],
            rows_v.at[0, pl.ds(0, TAIL)], gsem0).wait()
        s16 = scale_v[pl.ds(NFULL * G, LANES)]
        for t in range(TAIL):
            s = s16[t]
            for q in range(D // LANES):
                sl = pl.ds(q * LANES, LANES)
                rows_v[0, t, sl] = rows_v[0, t, sl] * s
        pltpu.sync_copy(rows_v.at[0, pl.ds(0, TAIL)], acc.at[tdst_v],
                        add=True)

        plsc.subcore_barrier()

        for k in range(ROWS_PER_SUB // 128):
            r0 = sid * ROWS_PER_SUB + k * 128
            pltpu.sync_copy(acc.at[pl.ds(r0, 128)],
                            out_hbm.at[cid, pl.ds(r0, 128)])

    return body(x, scale, src, dst)


def kernel(x, edge_filter, weight, edge_src, edge_dst):
    scale = _compute_scale(edge_filter.T, weight.T)
    partial = _sc_main(x, scale,
                       edge_src.astype(jnp.int32), edge_dst.astype(jnp.int32))
    return _combine(partial)


# speedup vs baseline: 1.1551x; 1.1551x over previous
import functools

import jax
import jax.numpy as jnp
from jax import lax
from jax.experimental import pallas as pl
from jax.experimental.pallas import tpu as pltpu
from jax.experimental.pallas import tpu_sc as plsc

N = 10000
E = 320000
D = 128
F = 16

NC = 2
NS = 16
NW = NC * NS

EPT = E // NW
G = 48
NFULL = EPT // G
TAIL = EPT - NFULL * G
NTRIP = 69
NPAD = 10112
ROWS_PER_SUB = NPAD // NS
LANES = 16


BX = 16000


def _scale_body(f_ref, w_ref, o_ref):
    i = pl.program_id(0)
    s = jnp.sum(f_ref[...], axis=0) * jnp.sum(w_ref[...], axis=0)
    o_ref[pl.ds(i * BX, BX)] = s


def _compute_scale(edge_filter_t, weight_t):
    return pl.pallas_call(
        _scale_body,
        grid=(E // BX,),
        in_specs=[
            pl.BlockSpec((F, BX), lambda i: (0, i)),
            pl.BlockSpec((F, BX), lambda i: (0, i)),
        ],
        out_specs=pl.BlockSpec((E,), lambda i: (0,)),
        out_shape=jax.ShapeDtypeStruct((E,), jnp.float32),
    )(edge_filter_t, weight_t)


def _add_body(p_ref, o_ref):
    o_ref[...] = p_ref[0] + p_ref[1]


def _combine(partial):
    BR = 2000
    return pl.pallas_call(
        _add_body,
        grid=(N // BR,),
        in_specs=[pl.BlockSpec((NC, BR, D), lambda i: (0, i, 0))],
        out_specs=pl.BlockSpec((BR, D), lambda i: (i, 0)),
        out_shape=jax.ShapeDtypeStruct((N, D), jnp.float32),
    )(partial)


def _sc_main(x, scale, src, dst):
    mesh = plsc.VectorSubcoreMesh(core_axis_name="c", subcore_axis_name="s")

    @functools.partial(
        pl.kernel,
        out_type=jax.ShapeDtypeStruct((NC, NPAD, D), jnp.float32),
        mesh=mesh,
        scratch_types=[
            pltpu.VMEM((EPT,), jnp.int32),
            pltpu.VMEM((EPT,), jnp.int32),
            pltpu.VMEM((EPT,), jnp.float32),
            pltpu.VMEM((3, G, D), jnp.float32),
            pltpu.VMEM((3, G), jnp.int32),
            pltpu.VMEM((TAIL,), jnp.int32),
            pltpu.VMEM_SHARED((NPAD, D), jnp.float32),
            pltpu.SemaphoreType.DMA,
            pltpu.SemaphoreType.DMA,
            pltpu.SemaphoreType.DMA,
            pltpu.SemaphoreType.DMA,
            pltpu.SemaphoreType.DMA,
            pltpu.SemaphoreType.DMA,
        ],
    )
    def body(x_hbm, scale_hbm, src_hbm, dst_hbm, out_hbm,
             src_v, dst_v, scale_v, rows_v, dstg_v, tdst_v, acc,
             gsem0, gsem1, gsem2, ssem0, ssem1, ssem2):
        cid = lax.axis_index("c")
        sid = lax.axis_index("s")
        wid = cid * NS + sid
        base = wid * EPT
        gsems = [gsem0, gsem1, gsem2]
        ssems = [ssem0, ssem1, ssem2]

        pltpu.sync_copy(src_hbm.at[pl.ds(base, EPT)], src_v)
        pltpu.sync_copy(dst_hbm.at[pl.ds(base, EPT)], dst_v)
        pltpu.sync_copy(scale_hbm.at[pl.ds(base, EPT)], scale_v)

        zero = jnp.zeros((LANES,), jnp.float32)

        def zrow(i, carry):
            for q in range(D // LANES):
                rows_v[0, i, pl.ds(q * LANES, LANES)] = zero
            return carry

        lax.fori_loop(0, G, zrow, 0)
        for k in range(ROWS_PER_SUB // G):
            pltpu.sync_copy(
                rows_v.at[0],
                acc.at[pl.ds(sid * ROWS_PER_SUB + k * G, G)])
        pltpu.sync_copy(
            rows_v.at[0, pl.ds(0, 8)],
            acc.at[pl.ds(sid * ROWS_PER_SUB + (ROWS_PER_SUB // G) * G, 8)])
        plsc.subcore_barrier()

        def issue_gather(j, b):
            return pltpu.async_copy(
                x_hbm.at[src_v.at[pl.ds(j * G, G)]], rows_v.at[b], gsems[b])

        def wait_gather(b):
            pltpu.make_async_copy(
                x_hbm.at[src_v.at[pl.ds(0, G)]], rows_v.at[b],
                gsems[b]).wait()

        def issue_scatter(b):
            return pltpu.async_copy(
                rows_v.at[b], acc.at[dstg_v.at[b]], ssems[b], add=True)

        def wait_scatter(b):
            pltpu.make_async_copy(
                rows_v.at[b], acc.at[dstg_v.at[b]], ssems[b]).wait()

        def stage_dst(j, b):
            for g in range(G // LANES):
                dstg_v[b, pl.ds(g * LANES, LANES)] = (
                    dst_v[pl.ds(j * G + g * LANES, LANES)])

        def compute(j, b):
            for g in range(G // LANES):
                s16 = scale_v[pl.ds(j * G + g * LANES, LANES)]
                for t in range(LANES):
                    e = g * LANES + t
                    s = s16[t]
                    for q in range(D // LANES):
                        sl = pl.ds(q * LANES, LANES)
                        rows_v[b, e, sl] = rows_v[b, e, sl] * s

        for b in range(2):
            stage_dst(b, b)
            issue_gather(b, b)

        def triple_body(p, carry):
            j0 = p * 3
            for b in range(3):
                j = j0 + b
                wait_gather(b)
                compute(j, b)
                issue_scatter(b)

                bn = (b + 2) % 3
                @pl.when(j > 0)
                def _():
                    wait_scatter(bn)

                @pl.when(j + 2 < NFULL)
                def _():
                    stage_dst(j + 2, bn)
                    issue_gather(j + 2, bn)
            return carry

        lax.fori_loop(0, NTRIP, triple_body, 0)

        jl = NFULL - 1
        wait_gather(0)
        compute(jl, 0)
        issue_scatter(0)
        wait_scatter(2)
        wait_scatter(0)

        tdst_v[...] = dst_v[pl.ds(NFULL * G, TAIL)]
        pltpu.async_copy(
            x_hbm.at[src_v.at[pl.ds(NFULL * G, TAIL)]],
            rows_v.at[1, pl.ds(0, TAIL)], gsem1).wait()
        s16 = scale_v[pl.ds(NFULL * G, LANES)]
        for t in range(TAIL):
            s = s16[t]
            for q in range(D // LANES):
                sl = pl.ds(q * LANES, LANES)
                rows_v[1, t, sl] = rows_v[1, t, sl] * s
        pltpu.sync_copy(rows_v.at[1, pl.ds(0, TAIL)], acc.at[tdst_v],
                        add=True)

        plsc.subcore_barrier()

        for k in range(4):
            r0 = sid * ROWS_PER_SUB + k * 128
            pltpu.sync_copy(acc.at[pl.ds(r0, 128)],
                            out_hbm.at[cid, pl.ds(r0, 128)])
        r0 = sid * ROWS_PER_SUB + 512
        pltpu.sync_copy(acc.at[pl.ds(r0, 120)],
                        out_hbm.at[cid, pl.ds(r0, 120)])

    return body(x, scale, src, dst)


def kernel(x, edge_filter, weight, edge_src, edge_dst):
    scale = _compute_scale(edge_filter.T, weight.T)
    partial = _sc_main(x, scale,
                       edge_src.astype(jnp.int32), edge_dst.astype(jnp.int32))
    return _combine(partial)


# speedup vs baseline: 1.1764x; 1.0184x over previous
import functools

import jax
import jax.numpy as jnp
from jax import lax
from jax.experimental import pallas as pl
from jax.experimental.pallas import tpu as pltpu
from jax.experimental.pallas import tpu_sc as plsc

N = 10000
E = 320000
D = 128
F = 16

NC = 2
NS = 16
NW = NC * NS

EPT = E // NW
G = 48
NFULL = EPT // G
TAIL = EPT - NFULL * G
NTRIP = 69
NPAD = 10112
ROWS_PER_SUB = NPAD // NS
LANES = 16


BX = 16000


def _scale_body(f_ref, w_ref, o_ref):
    i = pl.program_id(0)
    s = jnp.sum(f_ref[...], axis=0) * jnp.sum(w_ref[...], axis=0)
    o_ref[pl.ds(i * BX, BX)] = s


def _compute_scale(edge_filter_t, weight_t):
    return pl.pallas_call(
        _scale_body,
        grid=(E // BX,),
        in_specs=[
            pl.BlockSpec((F, BX), lambda i: (0, i)),
            pl.BlockSpec((F, BX), lambda i: (0, i)),
        ],
        out_specs=pl.BlockSpec((E,), lambda i: (0,)),
        out_shape=jax.ShapeDtypeStruct((E,), jnp.float32),
    )(edge_filter_t, weight_t)


def _add_body(p_ref, o_ref):
    o_ref[...] = p_ref[0] + p_ref[1]


def _combine(partial):
    BR = 2000
    return pl.pallas_call(
        _add_body,
        grid=(N // BR,),
        in_specs=[pl.BlockSpec((NC, BR, D), lambda i: (0, i, 0))],
        out_specs=pl.BlockSpec((BR, D), lambda i: (i, 0)),
        out_shape=jax.ShapeDtypeStruct((N, D), jnp.float32),
    )(partial)


def _sc_main(x, scale, src, dst):
    mesh = plsc.VectorSubcoreMesh(core_axis_name="c", subcore_axis_name="s")

    @functools.partial(
        pl.kernel,
        out_type=jax.ShapeDtypeStruct((NC, NPAD, D), jnp.float32),
        mesh=mesh,
        scratch_types=[
            pltpu.VMEM((EPT,), jnp.int32),
            pltpu.VMEM((EPT,), jnp.int32),
            pltpu.VMEM((EPT,), jnp.float32),
            pltpu.VMEM((3, G, D), jnp.float32),
            pltpu.VMEM((3, G), jnp.int32),
            pltpu.VMEM((TAIL,), jnp.int32),
            pltpu.VMEM_SHARED((NPAD, D), jnp.float32),
            pltpu.SemaphoreType.DMA,
            pltpu.SemaphoreType.DMA,
            pltpu.SemaphoreType.DMA,
            pltpu.SemaphoreType.DMA,
            pltpu.SemaphoreType.DMA,
            pltpu.SemaphoreType.DMA,
        ],
    )
    def body(x_hbm, scale_hbm, src_hbm, dst_hbm, out_hbm,
             src_v, dst_v, scale_v, rows_v, dstg_v, tdst_v, acc,
             gsem0, gsem1, gsem2, ssem0, ssem1, ssem2):
        cid = lax.axis_index("c")
        sid = lax.axis_index("s")
        wid = cid * NS + sid
        base = wid * EPT
        gsems = [gsem0, gsem1, gsem2]
        ssems = [ssem0, ssem1, ssem2]

        pltpu.async_copy(src_hbm.at[pl.ds(base, EPT)], src_v, gsem0)
        pltpu.async_copy(dst_hbm.at[pl.ds(base, EPT)], dst_v, gsem1)
        pltpu.async_copy(scale_hbm.at[pl.ds(base, EPT)], scale_v, gsem2)

        zero = jnp.zeros((LANES,), jnp.float32)

        def zrow(i, carry):
            for q in range(D // LANES):
                rows_v[0, i, pl.ds(q * LANES, LANES)] = zero
            return carry

        lax.fori_loop(0, G, zrow, 0)
        for k in range(ROWS_PER_SUB // G):
            pltpu.sync_copy(
                rows_v.at[0],
                acc.at[pl.ds(sid * ROWS_PER_SUB + k * G, G)])
        pltpu.sync_copy(
            rows_v.at[0, pl.ds(0, 8)],
            acc.at[pl.ds(sid * ROWS_PER_SUB + (ROWS_PER_SUB // G) * G, 8)])
        pltpu.make_async_copy(src_hbm.at[pl.ds(base, EPT)], src_v,
                              gsem0).wait()
        pltpu.make_async_copy(dst_hbm.at[pl.ds(base, EPT)], dst_v,
                              gsem1).wait()
        pltpu.make_async_copy(scale_hbm.at[pl.ds(base, EPT)], scale_v,
                              gsem2).wait()
        plsc.subcore_barrier()

        def issue_gather(j, b):
            return pltpu.async_copy(
                x_hbm.at[src_v.at[pl.ds(j * G, G)]], rows_v.at[b], gsems[b])

        def wait_gather(b):
            pltpu.make_async_copy(
                x_hbm.at[src_v.at[pl.ds(0, G)]], rows_v.at[b],
                gsems[b]).wait()

        def issue_scatter(b):
            return pltpu.async_copy(
                rows_v.at[b], acc.at[dstg_v.at[b]], ssems[b], add=True)

        def wait_scatter(b):
            pltpu.make_async_copy(
                rows_v.at[b], acc.at[dstg_v.at[b]], ssems[b]).wait()

        def stage_dst(j, b):
            for g in range(G // LANES):
                dstg_v[b, pl.ds(g * LANES, LANES)] = (
                    dst_v[pl.ds(j * G + g * LANES, LANES)])

        def compute(j, b):
            for g in range(G // LANES):
                s16 = scale_v[pl.ds(j * G + g * LANES, LANES)]
                for t in range(LANES):
                    e = g * LANES + t
                    s = s16[t]
                    for q in range(D // LANES):
                        sl = pl.ds(q * LANES, LANES)
                        rows_v[b, e, sl] = rows_v[b, e, sl] * s

        for b in range(2):
            stage_dst(b, b)
            issue_gather(b, b)

        def triple_body(p, carry):
            j0 = p * 3
            for b in range(3):
                j = j0 + b
                wait_gather(b)
                compute(j, b)
                issue_scatter(b)

                bn = (b + 2) % 3
                @pl.when(j > 0)
                def _():
                    wait_scatter(bn)

                @pl.when(j + 2 < NFULL)
                def _():
                    stage_dst(j + 2, bn)
                    issue_gather(j + 2, bn)
            return carry

        lax.fori_loop(0, NTRIP, triple_body, 0)

        jl = NFULL - 1
        wait_gather(0)
        compute(jl, 0)
        issue_scatter(0)
        wait_scatter(2)
        wait_scatter(0)

        tdst_v[...] = dst_v[pl.ds(NFULL * G, TAIL)]
        pltpu.async_copy(
            x_hbm.at[src_v.at[pl.ds(NFULL * G, TAIL)]],
            rows_v.at[1, pl.ds(0, TAIL)], gsem1).wait()
        s16 = scale_v[pl.ds(NFULL * G, LANES)]
        for t in range(TAIL):
            s = s16[t]
            for q in range(D // LANES):
                sl = pl.ds(q * LANES, LANES)
                rows_v[1, t, sl] = rows_v[1, t, sl] * s
        pltpu.sync_copy(rows_v.at[1, pl.ds(0, TAIL)], acc.at[tdst_v],
                        add=True)

        plsc.subcore_barrier()

        for k in range(4):
            r0 = sid * ROWS_PER_SUB + k * 128
            pltpu.sync_copy(acc.at[pl.ds(r0, 128)],
                            out_hbm.at[cid, pl.ds(r0, 128)])
        r0 = sid * ROWS_PER_SUB + 512
        pltpu.sync_copy(acc.at[pl.ds(r0, 120)],
                        out_hbm.at[cid, pl.ds(r0, 120)])

    return body(x, scale, src, dst)


def kernel(x, edge_filter, weight, edge_src, edge_dst):
    scale = _compute_scale(edge_filter.T, weight.T)
    partial = _sc_main(x, scale,
                       edge_src.astype(jnp.int32), edge_dst.astype(jnp.int32))
    return _combine(partial)
